# Initial kernel scaffold; baseline (speedup 1.0000x reference)
import jax
import jax.numpy as jnp
from jax.experimental import pallas as pl


def kernel(q1, k1, q2, k2, q3, k3, temp_vid, temp_aud, temp_ocr):
    raise NotImplementedError("write your pallas kernel here")


# speedup vs baseline: 5.2996x; 5.2996x over previous
import functools

import jax
import jax.numpy as jnp
from jax import lax
from jax.experimental import pallas as pl
from jax.experimental.pallas import tpu as pltpu

B, N, D = 32, 4096, 1024
TOP_K = 25
NBLK = 8
BLK = N // NBLK


def _tc_body(gate_ref, q_ref, k_ref, out_ref, s_ref):
    step = pl.program_id(0)

    q = q_ref[...]
    k = k_ref[...]
    qh = q / jnp.maximum(
        jnp.sqrt(jnp.sum(q * q, axis=1, keepdims=True)), 1e-12
    )
    kh = k / jnp.maximum(
        jnp.sqrt(jnp.sum(k * k, axis=1, keepdims=True)), 1e-12
    )
    qn = jnp.maximum(jnp.sqrt(jnp.sum(qh * qh, axis=1, keepdims=True)), 1e-8)
    kn = jnp.maximum(jnp.sqrt(jnp.sum(kh * kh, axis=1, keepdims=True)), 1e-8)
    dot = jax.lax.dot_general(
        qh, kh, (((1,), (1,)), ((), ())), preferred_element_type=jnp.float32
    )
    cos = dot / (qn * kn.reshape(1, BLK))
    s_ref[:, pl.ds(step * BLK, BLK)] = jnp.clip(cos, 0.0, 1.0)

    @pl.when(step == NBLK - 1)
    def _finish():
        s = s_ref[...]
        m = jnp.max(s, axis=1, keepdims=True)
        e = jnp.exp(s - m)
        attn = e / jnp.sum(e, axis=1, keepdims=True)

        iota = lax.broadcasted_iota(jnp.int32, (B, N), 1)
        work = s
        union = jnp.zeros((1, N), dtype=jnp.float32)
        selfset = jnp.zeros((1, N), dtype=jnp.float32)
        for t in range(TOP_K):
            mx = jnp.max(work, axis=1, keepdims=True)
            idx = jnp.min(
                jnp.where(work == mx, iota, N), axis=1, keepdims=True
            )
            sel = (iota == idx).astype(jnp.float32)
            hit = jnp.max(sel, axis=0, keepdims=True)
            union = jnp.maximum(union, hit)
            if t == 0:
                selfset = hit
            work = jnp.where(iota == idx, -1.0, work)

        mask = union * (1.0 - selfset)
        out_ref[...] = attn * (gate_ref[0] * mask)


def _tc_call(gate, q1, k1):
    return pl.pallas_call(
        _tc_body,
        grid=(NBLK,),
        in_specs=[
            pl.BlockSpec(memory_space=pltpu.SMEM),
            pl.BlockSpec((B, D), lambda i: (0, 0)),
            pl.BlockSpec((BLK, D), lambda i: (i, 0)),
        ],
        out_specs=pl.BlockSpec((B, N), lambda i: (0, 0)),
        out_shape=jax.ShapeDtypeStruct((B, N), jnp.float32),
        scratch_shapes=[pltpu.VMEM((B, N), jnp.float32)],
    )(gate, q1, k1)


@jax.jit
def kernel(q1, k1, q2, k2, q3, k3, temp_vid, temp_aud, temp_ocr):
    del q2, k2, q3, k3, temp_aud, temp_ocr
    gate = jax.nn.sigmoid(temp_vid) * 2.0
    return _tc_call(gate, q1, k1)
